# baseline (speedup 1.0000x reference)
import functools

import jax
import jax.numpy as jnp
from jax import lax
from jax.experimental import pallas as pl
from jax.experimental.pallas import tpu as pltpu
from jax.experimental.pallas import tpu_sc as plsc

_N = 10000
_E = 320000
_D = 128
_R = 8

_NC = 2
_NS = 16
_NW = _NC * _NS
_EPW = _E // _NW
_C = 80
_NCHUNK = _EPW // _C
_RG = 3
_RD = 2
_RI = 6
_RPS = 624
_TAIL = _N - _NS * _RPS
_ZR = 24

_BN = 1000
_NB = _N // _BN


def _mm_body(x_ref, w_ref, o_ref):
    o_ref[0] = jnp.dot(x_ref[...], w_ref[0], preferred_element_type=jnp.float32)


def _transform(x, W):
    return pl.pallas_call(
        _mm_body,
        grid=(_NB, _R),
        in_specs=[
            pl.BlockSpec((_BN, _D), lambda i, r: (i, 0)),
            pl.BlockSpec((1, _D, _D), lambda i, r: (r, 0, 0)),
        ],
        out_specs=pl.BlockSpec((1, _BN, _D), lambda i, r: (r, i, 0)),
        out_shape=jax.ShapeDtypeStruct((_R, _N, _D), jnp.float32),
    )(x, W)


def _mm_relu_body(p_ref, w_ref, o_ref):
    h = jnp.maximum(p_ref[0] + p_ref[1], 0.0)
    o_ref[0] = jnp.dot(h, w_ref[0], preferred_element_type=jnp.float32)


def _transform_relu(partials, W):
    return pl.pallas_call(
        _mm_relu_body,
        grid=(_NB, _R),
        in_specs=[
            pl.BlockSpec((2, _BN, _D), lambda i, r: (0, i, 0)),
            pl.BlockSpec((1, _D, _D), lambda i, r: (r, 0, 0)),
        ],
        out_specs=pl.BlockSpec((1, _BN, _D), lambda i, r: (r, i, 0)),
        out_shape=jax.ShapeDtypeStruct((_R, _N, _D), jnp.float32),
    )(partials, W)


def _add_body(p_ref, o_ref):
    o_ref[...] = p_ref[0] + p_ref[1]


def _combine(partials):
    return pl.pallas_call(
        _add_body,
        grid=(_NB,),
        in_specs=[pl.BlockSpec((2, _BN, _D), lambda i: (0, i, 0))],
        out_specs=pl.BlockSpec((_BN, _D), lambda i: (i, 0)),
        out_shape=jax.ShapeDtypeStruct((_N, _D), jnp.float32),
    )(partials)


def _sc_body(table, meta, nrmx, out, acc, metab, nrmxb, grow, srow, isem,
             nsem, gsem, ssem, zsem):
    c = lax.axis_index("c")
    s = lax.axis_index("s")
    wid = s * _NC + c

    zero = jnp.zeros((16,), jnp.float32)
    for i in range(_ZR):
        for j in range(_D // 16):
            grow[0][i, pl.ds(j * 16, 16)] = zero
    zblk = grow[0].at[pl.ds(0, _ZR)]
    for t in range(_RPS // _ZR):
        pltpu.async_copy(zblk, acc.at[pl.ds(s * _RPS + t * _ZR, _ZR)], zsem)

    @pl.when(s == 0)
    def _zero_tail():
        pltpu.sync_copy(grow[0].at[pl.ds(0, _TAIL)],
                        acc.at[pl.ds(_NS * _RPS, _TAIL)])

    for t in range(_RPS // _ZR):
        pltpu.make_async_copy(zblk, acc.at[pl.ds(0, _ZR)], zsem).wait()
    plsc.subcore_barrier()


    def start_idx(row, g):
        pltpu.async_copy(meta.at[wid, g], metab.at[row], isem[row])

    def wait_idx(row):
        pltpu.make_async_copy(meta.at[wid, 0], metab.at[row], isem[row]).wait()

    def _nrm_slc(bb):
        return nrmxb.at[pl.ds(bb * _C * 16, _C * 16)]

    def start_nrm(bb, g):
        pltpu.async_copy(nrmx.at[wid, g], _nrm_slc(bb), nsem[bb])

    def wait_nrm(bb):
        pltpu.make_async_copy(nrmx.at[wid, 0], _nrm_slc(bb), nsem[bb]).wait()

    def start_gather(b, row):
        pltpu.async_copy(table.at[metab.at[row, 0]], grow[b], gsem[b])

    def wait_gather(b, row):
        pltpu.make_async_copy(table.at[metab.at[row, 0]], grow[b],
                              gsem[b]).wait()

    def start_scatter(row):
        pltpu.async_copy(srow, acc.at[metab.at[row, 1]], ssem, add=True)

    def wait_scatter(row):
        pltpu.make_async_copy(srow, acc.at[metab.at[row, 1]], ssem).wait()

    def scale(b, bb):
        base = bb * _C * 16

        def grp(k, carry):
            for l in range(16):
                e = k * 16 + l
                nb = nrmxb[pl.ds(base + e * 16, 16)]
                for j in range(_D // 16):
                    sl = pl.ds(j * 16, 16)
                    srow[e, sl] = grow[b][e, sl] * nb
            return carry
        lax.fori_loop(0, _C // 16, grp, 0)

    for g0 in range(_RI - 2):
        start_idx(g0, g0)
    for b in range(2):
        start_nrm(b, b)
    for b in range(2):
        wait_idx(b)
        start_gather(b, b)

    def super_iter(outer, carry):
        for pos in range(_RI):
            g = outer * _RI + pos
            b = pos % _RG
            bb = pos % _RD
            row = pos

            @pl.when(g < _NCHUNK)
            def _chunk():
                wait_gather(b, row)

                @pl.when(g >= 1)
                def _drain_prev():
                    wait_scatter((pos - 1) % _RI)

                @pl.when(g + _RI - 2 < _NCHUNK)
                def _prefetch():
                    start_idx((pos - 2) % _RI, g + _RI - 2)

                @pl.when(g + 2 < _NCHUNK)
                def _next_gather():
                    wait_idx((pos + 2) % _RI)
                    start_gather((pos + 2) % _RG, (pos + 2) % _RI)

                wait_nrm(bb)
                scale(b, bb)
                start_scatter(row)

                @pl.when(g + 2 < _NCHUNK)
                def _next_nrm():
                    start_nrm(bb, g + 2)

        return carry

    lax.fori_loop(0, pl.cdiv(_NCHUNK, _RI), super_iter, 0)

    wait_scatter((_NCHUNK - 1) % _RI)

    plsc.subcore_barrier()
    pltpu.sync_copy(acc.at[pl.ds(s * _RPS, _RPS)],
                    out.at[c, pl.ds(s * _RPS, _RPS)])

    @pl.when(s == 0)
    def _write_tail():
        pltpu.sync_copy(acc.at[pl.ds(_NS * _RPS, _TAIL)],
                        out.at[c, pl.ds(_NS * _RPS, _TAIL)])


def _sc_agg(table2d, meta, nrmx):
    mesh = plsc.VectorSubcoreMesh(core_axis_name="c", subcore_axis_name="s")
    f = pl.kernel(
        _sc_body,
        out_type=jax.ShapeDtypeStruct((_NC, _N, _D), jnp.float32),
        mesh=mesh,
        scratch_types=[
            pltpu.VMEM_SHARED((_N, _D), jnp.float32),
            pltpu.VMEM((_RI, 2, _C), jnp.int32),
            pltpu.VMEM((_RD * _C * 16,), jnp.float32),
            [pltpu.VMEM((_C, _D), jnp.float32)] * _RG,
            pltpu.VMEM((_C, _D), jnp.float32),
            [pltpu.SemaphoreType.DMA] * _RI,
            [pltpu.SemaphoreType.DMA] * _RD,
            [pltpu.SemaphoreType.DMA] * _RG,
            pltpu.SemaphoreType.DMA,
            pltpu.SemaphoreType.DMA,
        ],
    )
    return f(table2d, meta, nrmx)


@jax.jit
def kernel(emb, edge_index, etypes, norm, W1, W2):
    src = edge_index[0].astype(jnp.int32)
    dstv = edge_index[1].astype(jnp.int32)
    fidx = (etypes.astype(jnp.int32) * _N + src).reshape(_NW, _NCHUNK, _C)
    dst3 = dstv.reshape(_NW, _NCHUNK, _C)
    meta = jnp.stack([fidx, dst3], axis=2)
    nrmx = jnp.broadcast_to(
        norm[:, 0].reshape(_NW, _NCHUNK, _C)[..., None],
        (_NW, _NCHUNK, _C, 16)).reshape(_NW, _NCHUNK, _C * 16)

    t1 = _transform(emb, W1).reshape(_R * _N, _D)
    p1 = _sc_agg(t1, meta, nrmx)
    t2 = _transform_relu(p1, W2).reshape(_R * _N, _D)
    p2 = _sc_agg(t2, meta, nrmx)
    return _combine(p2)


# speedup vs baseline: 1.1072x; 1.1072x over previous
import functools

import jax
import jax.numpy as jnp
from jax import lax
from jax.experimental import pallas as pl
from jax.experimental.pallas import tpu as pltpu
from jax.experimental.pallas import tpu_sc as plsc

_N = 10000
_E = 320000
_D = 128
_R = 8

_NC = 2
_NS = 16
_NW = _NC * _NS
_EPW = _E // _NW
_C = 80
_NCHUNK = _EPW // _C
_RG = 4
_RD = 2
_RI = 6
_POS = 12
_RPS = 624
_TAIL = _N - _NS * _RPS
_ZR = 24

_BN = 1000
_NB = _N // _BN


def _mm_body(x_ref, w_ref, o_ref):
    o_ref[0] = jnp.dot(x_ref[...], w_ref[0], preferred_element_type=jnp.float32)


def _transform(x, W):
    return pl.pallas_call(
        _mm_body,
        grid=(_NB, _R),
        in_specs=[
            pl.BlockSpec((_BN, _D), lambda i, r: (i, 0)),
            pl.BlockSpec((1, _D, _D), lambda i, r: (r, 0, 0)),
        ],
        out_specs=pl.BlockSpec((1, _BN, _D), lambda i, r: (r, i, 0)),
        out_shape=jax.ShapeDtypeStruct((_R, _N, _D), jnp.float32),
    )(x, W)


def _mm_relu_body(p_ref, w_ref, o_ref):
    h = jnp.maximum(p_ref[0] + p_ref[1], 0.0)
    o_ref[0] = jnp.dot(h, w_ref[0], preferred_element_type=jnp.float32)


def _transform_relu(partials, W):
    return pl.pallas_call(
        _mm_relu_body,
        grid=(_NB, _R),
        in_specs=[
            pl.BlockSpec((2, _BN, _D), lambda i, r: (0, i, 0)),
            pl.BlockSpec((1, _D, _D), lambda i, r: (r, 0, 0)),
        ],
        out_specs=pl.BlockSpec((1, _BN, _D), lambda i, r: (r, i, 0)),
        out_shape=jax.ShapeDtypeStruct((_R, _N, _D), jnp.float32),
    )(partials, W)


def _add_body(p_ref, o_ref):
    o_ref[...] = p_ref[0] + p_ref[1]


def _combine(partials):
    return pl.pallas_call(
        _add_body,
        grid=(_NB,),
        in_specs=[pl.BlockSpec((2, _BN, _D), lambda i: (0, i, 0))],
        out_specs=pl.BlockSpec((_BN, _D), lambda i: (i, 0)),
        out_shape=jax.ShapeDtypeStruct((_N, _D), jnp.float32),
    )(partials)


def _sc_body(table, meta, nrmx, out, acc, metab, nrmxb, grow, isem,
             nsem, gsem, ssem, zsem):
    c = lax.axis_index("c")
    s = lax.axis_index("s")
    wid = s * _NC + c

    zero = jnp.zeros((16,), jnp.float32)
    for i in range(_ZR):
        for j in range(_D // 16):
            grow[0][i, pl.ds(j * 16, 16)] = zero
    zblk = grow[0].at[pl.ds(0, _ZR)]
    for t in range(_RPS // _ZR):
        pltpu.async_copy(zblk, acc.at[pl.ds(s * _RPS + t * _ZR, _ZR)], zsem)

    @pl.when(s == 0)
    def _zero_tail():
        pltpu.sync_copy(grow[0].at[pl.ds(0, _TAIL)],
                        acc.at[pl.ds(_NS * _RPS, _TAIL)])

    for t in range(_RPS // _ZR):
        pltpu.make_async_copy(zblk, acc.at[pl.ds(0, _ZR)], zsem).wait()
    plsc.subcore_barrier()


    def start_idx(row, g):
        pltpu.async_copy(meta.at[wid, g], metab.at[row], isem[row])

    def wait_idx(row):
        pltpu.make_async_copy(meta.at[wid, 0], metab.at[row], isem[row]).wait()

    def _nrm_slc(bb):
        return nrmxb.at[pl.ds(bb * _C * 16, _C * 16)]

    def start_nrm(bb, g):
        pltpu.async_copy(nrmx.at[wid, g], _nrm_slc(bb), nsem[bb])

    def wait_nrm(bb):
        pltpu.make_async_copy(nrmx.at[wid, 0], _nrm_slc(bb), nsem[bb]).wait()

    def start_gather(b, row):
        pltpu.async_copy(table.at[metab.at[row, 0]], grow[b], gsem[b])

    def wait_gather(b, row):
        pltpu.make_async_copy(table.at[metab.at[row, 0]], grow[b],
                              gsem[b]).wait()

    def start_scatter(b, sp, row):
        pltpu.async_copy(grow[b], acc.at[metab.at[row, 1]], ssem[sp],
                         add=True)

    def wait_scatter(sp, row):
        pltpu.make_async_copy(grow[0], acc.at[metab.at[row, 1]],
                              ssem[sp]).wait()

    def scale(b, bb):
        base = bb * _C * 16

        def grp(k, carry):
            for l in range(16):
                e = k * 16 + l
                nb = nrmxb[pl.ds(base + e * 16, 16)]
                for j in range(_D // 16):
                    sl = pl.ds(j * 16, 16)
                    grow[b][e, sl] = grow[b][e, sl] * nb
            return carry
        lax.fori_loop(0, _C // 16, grp, 0)

    for g0 in range(_RI - 2):
        start_idx(g0, g0)
    for b in range(2):
        start_nrm(b, b)
    for b in range(2):
        wait_idx(b)
        start_gather(b, b)

    def super_iter(outer, carry):
        for pos in range(_POS):
            g = outer * _POS + pos
            b = pos % _RG
            bb = pos % _RD
            row = pos % _RI

            @pl.when(g < _NCHUNK)
            def _chunk():
                wait_gather(b, row)

                @pl.when(g >= 2)
                def _drain_prev():
                    wait_scatter(bb, (pos - 2) % _RI)

                @pl.when(g + _RI - 2 < _NCHUNK)
                def _prefetch():
                    start_idx((pos - 2) % _RI, g + _RI - 2)

                @pl.when(g + 2 < _NCHUNK)
                def _next_gather():
                    wait_idx((pos + 2) % _RI)
                    start_gather((pos + 2) % _RG, (pos + 2) % _RI)

                wait_nrm(bb)
                scale(b, bb)
                start_scatter(b, bb, row)

                @pl.when(g + 2 < _NCHUNK)
                def _next_nrm():
                    start_nrm(bb, g + 2)

        return carry

    lax.fori_loop(0, pl.cdiv(_NCHUNK, _POS), super_iter, 0)

    for gg in (_NCHUNK - 2, _NCHUNK - 1):
        wait_scatter(gg % _RD, gg % _RI)

    plsc.subcore_barrier()
    pltpu.sync_copy(acc.at[pl.ds(s * _RPS, _RPS)],
                    out.at[c, pl.ds(s * _RPS, _RPS)])

    @pl.when(s == 0)
    def _write_tail():
        pltpu.sync_copy(acc.at[pl.ds(_NS * _RPS, _TAIL)],
                        out.at[c, pl.ds(_NS * _RPS, _TAIL)])


def _sc_agg(table2d, meta, nrmx):
    mesh = plsc.VectorSubcoreMesh(core_axis_name="c", subcore_axis_name="s")
    f = pl.kernel(
        _sc_body,
        out_type=jax.ShapeDtypeStruct((_NC, _N, _D), jnp.float32),
        mesh=mesh,
        scratch_types=[
            pltpu.VMEM_SHARED((_N, _D), jnp.float32),
            pltpu.VMEM((_RI, 2, _C), jnp.int32),
            pltpu.VMEM((_RD * _C * 16,), jnp.float32),
            [pltpu.VMEM((_C, _D), jnp.float32)] * _RG,
            [pltpu.SemaphoreType.DMA] * _RI,
            [pltpu.SemaphoreType.DMA] * _RD,
            [pltpu.SemaphoreType.DMA] * _RG,
            [pltpu.SemaphoreType.DMA] * _RD,
            pltpu.SemaphoreType.DMA,
        ],
    )
    return f(table2d, meta, nrmx)


@jax.jit
def kernel(emb, edge_index, etypes, norm, W1, W2):
    src = edge_index[0].astype(jnp.int32)
    dstv = edge_index[1].astype(jnp.int32)
    fidx = (etypes.astype(jnp.int32) * _N + src).reshape(_NW, _NCHUNK, _C)
    dst3 = dstv.reshape(_NW, _NCHUNK, _C)
    meta = jnp.stack([fidx, dst3], axis=2)
    nrmx = jnp.broadcast_to(
        norm[:, 0].reshape(_NW, _NCHUNK, _C)[..., None],
        (_NW, _NCHUNK, _C, 16)).reshape(_NW, _NCHUNK, _C * 16)

    t1 = _transform(emb, W1).reshape(_R * _N, _D)
    p1 = _sc_agg(t1, meta, nrmx)
    t2 = _transform_relu(p1, W2).reshape(_R * _N, _D)
    p2 = _sc_agg(t2, meta, nrmx)
    return _combine(p2)


# speedup vs baseline: 1.1089x; 1.0016x over previous
import jax
import jax.numpy as jnp
from jax import lax
from jax.experimental import pallas as pl
from jax.experimental.pallas import tpu as pltpu
from jax.experimental.pallas import tpu_sc as plsc

_N = 10000
_E = 320000
_D = 128
_R = 8

_NC = 2
_NS = 16
_NW = _NC * _NS
_EPW = _E // _NW
_C = 80
_NCHUNK = _EPW // _C
_RG = 4
_RD = 2
_RI = 6
_POS = 12
_RPS = 624
_TAIL = _N - _NS * _RPS
_ZR = 24

_BN = 1000
_NB = _N // _BN


def _mm_body(x_ref, w_ref, o_ref):
    o_ref[0] = jnp.dot(x_ref[...], w_ref[0], preferred_element_type=jnp.float32)


def _transform(x, W):
    return pl.pallas_call(
        _mm_body,
        grid=(_NB, _R),
        in_specs=[
            pl.BlockSpec((_BN, _D), lambda i, r: (i, 0)),
            pl.BlockSpec((1, _D, _D), lambda i, r: (r, 0, 0)),
        ],
        out_specs=pl.BlockSpec((1, _BN, _D), lambda i, r: (r, i, 0)),
        out_shape=jax.ShapeDtypeStruct((_R, _N, _D), jnp.float32),
    )(x, W)


def _mm_relu_body(p_ref, w_ref, o_ref):
    h = jnp.maximum(p_ref[0] + p_ref[1], 0.0)
    o_ref[0] = jnp.dot(h, w_ref[0], preferred_element_type=jnp.float32)


def _transform_relu(partials, W):
    return pl.pallas_call(
        _mm_relu_body,
        grid=(_NB, _R),
        in_specs=[
            pl.BlockSpec((2, _BN, _D), lambda i, r: (0, i, 0)),
            pl.BlockSpec((1, _D, _D), lambda i, r: (r, 0, 0)),
        ],
        out_specs=pl.BlockSpec((1, _BN, _D), lambda i, r: (r, i, 0)),
        out_shape=jax.ShapeDtypeStruct((_R, _N, _D), jnp.float32),
    )(partials, W)


def _add_body(p_ref, o_ref):
    o_ref[...] = p_ref[0] + p_ref[1]


def _combine(partials):
    return pl.pallas_call(
        _add_body,
        grid=(_NB,),
        in_specs=[pl.BlockSpec((2, _BN, _D), lambda i: (0, i, 0))],
        out_specs=pl.BlockSpec((_BN, _D), lambda i: (i, 0)),
        out_shape=jax.ShapeDtypeStruct((_N, _D), jnp.float32),
    )(partials)


def _sc_body(table, meta, nrmx, out, acc, metab, nrmxb, grow, isem,
             nsem, gsem, ssem, zsem):
    c = lax.axis_index("c")
    s = lax.axis_index("s")
    wid = s * _NC + c

    zero = jnp.zeros((16,), jnp.float32)
    for i in range(_ZR):
        for j in range(_D // 16):
            grow[0][i, pl.ds(j * 16, 16)] = zero
    zblk = grow[0].at[pl.ds(0, _ZR)]
    for t in range(_RPS // _ZR):
        pltpu.async_copy(zblk, acc.at[pl.ds(s * _RPS + t * _ZR, _ZR)], zsem)

    @pl.when(s == 0)
    def _zero_tail():
        pltpu.sync_copy(grow[0].at[pl.ds(0, _TAIL)],
                        acc.at[pl.ds(_NS * _RPS, _TAIL)])

    for t in range(_RPS // _ZR):
        pltpu.make_async_copy(zblk, acc.at[pl.ds(0, _ZR)], zsem).wait()
    plsc.subcore_barrier()


    def start_idx(row, g):
        pltpu.async_copy(meta.at[wid, g], metab.at[row], isem[row])

    def wait_idx(row):
        pltpu.make_async_copy(meta.at[wid, 0], metab.at[row], isem[row]).wait()

    def _nrm_slc(bb):
        return nrmxb.at[pl.ds(bb * _C * 16, _C * 16)]

    def start_nrm(bb, g):
        pltpu.async_copy(nrmx.at[wid, g], _nrm_slc(bb), nsem[bb])

    def wait_nrm(bb):
        pltpu.make_async_copy(nrmx.at[wid, 0], _nrm_slc(bb), nsem[bb]).wait()

    def start_gather(b, row):
        pltpu.async_copy(table.at[metab.at[row, 0]], grow[b], gsem[b])

    def wait_gather(b, row):
        pltpu.make_async_copy(table.at[metab.at[row, 0]], grow[b],
                              gsem[b]).wait()

    def start_scatter(b, sp, row):
        pltpu.async_copy(grow[b], acc.at[metab.at[row, 1]], ssem[sp],
                         add=True)

    def wait_scatter(sp, row):
        pltpu.make_async_copy(grow[0], acc.at[metab.at[row, 1]],
                              ssem[sp]).wait()

    def scale(b, bb):
        base = bb * _C * 16

        def grp(k, carry):
            for l in range(16):
                e = k * 16 + l
                nb = nrmxb[pl.ds(base + e * 16, 16)]
                for j in range(_D // 16):
                    sl = pl.ds(j * 16, 16)
                    grow[b][e, sl] = grow[b][e, sl] * nb
            return carry
        lax.fori_loop(0, _C // 16, grp, 0)

    for g0 in range(_RI - 2):
        start_idx(g0, g0)
    for b in range(2):
        start_nrm(b, b)
    for b in range(2):
        wait_idx(b)
        start_gather(b, b)

    def super_iter(outer, carry):
        for pos in range(_POS):
            g = outer * _POS + pos
            b = pos % _RG
            bb = pos % _RD
            row = pos % _RI

            @pl.when(g < _NCHUNK)
            def _chunk():
                wait_gather(b, row)

                @pl.when(g >= 2)
                def _drain_prev():
                    wait_scatter(bb, (pos - 2) % _RI)

                @pl.when(g + _RI - 2 < _NCHUNK)
                def _prefetch():
                    start_idx((pos - 2) % _RI, g + _RI - 2)

                @pl.when(g + 2 < _NCHUNK)
                def _next_gather():
                    wait_idx((pos + 2) % _RI)
                    start_gather((pos + 2) % _RG, (pos + 2) % _RI)

                wait_nrm(bb)
                scale(b, bb)
                start_scatter(b, bb, row)

                @pl.when(g + 2 < _NCHUNK)
                def _next_nrm():
                    start_nrm(bb, g + 2)

        return carry

    lax.fori_loop(0, pl.cdiv(_NCHUNK, _POS), super_iter, 0)

    for gg in (_NCHUNK - 2, _NCHUNK - 1):
        wait_scatter(gg % _RD, gg % _RI)

    plsc.subcore_barrier()
    pltpu.sync_copy(acc.at[pl.ds(s * _RPS, _RPS)],
                    out.at[c, pl.ds(s * _RPS, _RPS)])

    @pl.when(s == 0)
    def _write_tail():
        pltpu.sync_copy(acc.at[pl.ds(_NS * _RPS, _TAIL)],
                        out.at[c, pl.ds(_NS * _RPS, _TAIL)])


def _sc_agg(table2d, meta, nrmx):
    mesh = plsc.VectorSubcoreMesh(core_axis_name="c", subcore_axis_name="s")
    f = pl.kernel(
        _sc_body,
        out_type=jax.ShapeDtypeStruct((_NC, _N, _D), jnp.float32),
        mesh=mesh,
        scratch_types=[
            pltpu.VMEM_SHARED((_N, _D), jnp.float32),
            pltpu.VMEM((_RI, 2, _C), jnp.int32),
            pltpu.VMEM((_RD * _C * 16,), jnp.float32),
            [pltpu.VMEM((_C, _D), jnp.float32)] * _RG,
            [pltpu.SemaphoreType.DMA] * _RI,
            [pltpu.SemaphoreType.DMA] * _RD,
            [pltpu.SemaphoreType.DMA] * _RG,
            [pltpu.SemaphoreType.DMA] * _RD,
            pltpu.SemaphoreType.DMA,
        ],
    )
    return f(table2d, meta, nrmx)


@jax.jit
def kernel(emb, edge_index, etypes, norm, W1, W2):
    src = edge_index[0].astype(jnp.int32)
    dstv = edge_index[1].astype(jnp.int32)
    fidx = (etypes.astype(jnp.int32) * _N + src).reshape(_NW, _NCHUNK, _C)
    dst3 = dstv.reshape(_NW, _NCHUNK, _C)
    meta = jnp.stack([fidx, dst3], axis=2)
    nrmx = jnp.broadcast_to(
        norm[:, 0].reshape(_NW, _NCHUNK, _C)[..., None],
        (_NW, _NCHUNK, _C, 16)).reshape(_NW, _NCHUNK, _C * 16)

    t1 = _transform(emb, W1).reshape(_R * _N, _D)
    p1 = _sc_agg(t1, meta, nrmx)
    t2 = _transform_relu(p1, W2).reshape(_R * _N, _D)
    p2 = _sc_agg(t2, meta, nrmx)
    return _combine(p2)
